# baseline (speedup 1.0000x reference)
import jax
import jax.numpy as jnp
from jax import lax
from jax.experimental import pallas as pl
from jax.experimental.pallas import tpu as pltpu
from jax.experimental.pallas import tpu_sc as plsc

_B = 8
_D = 512
_TOTAL = 15488
_NPOS = 8192
_ALPHA = float(_D) ** 0.5
_L = 16
_CHUNK = 32
_NCHUNKS = _TOTAL // _CHUNK
_NW = 32
_NMAX = -(-_NCHUNKS // _NW)


def _body(meta_hbm, emb_hbm, pos_hbm, out_hbm,
          meta_v, idx0, idx1, idx2, emb0, emb1, emb2, pos0, pos1, pos2,
          se0, se1, se2, sp0, sp1, sp2, so0, so1, so2):
  cid = lax.axis_index("c")
  sid = lax.axis_index("s")
  wid = sid * 2 + cid

  pltpu.sync_copy(meta_hbm, meta_v)
  off = [meta_v[b, :] for b in range(_B)]
  high = [meta_v[_B + b, :] for b in range(_B)]
  lanes = lax.iota(jnp.int32, _L)

  nloc = (_NCHUNKS - wid + _NW - 1) // _NW

  idxs = [idx0, idx1, idx2]
  embs = [emb0, emb1, emb2]
  poss = [pos0, pos1, pos2]
  ses = [se0, se1, se2]
  sps = [sp0, sp1, sp2]
  sos = [so0, so1, so2]

  def compute_idx(base, idx_ref):
    for g in range(_CHUNK // _L):
      t = base + g * _L + lanes
      off_s = off[0]
      high_s = high[0]
      for s in range(1, _B):
        m = t >= off[s]
        off_s = jnp.where(m, off[s], off_s)
        high_s = jnp.where(m, high[s], high_s)
      p = jnp.minimum(t - off_s, high_s)
      p = jnp.maximum(jnp.minimum(p, _NPOS - 1), 0)
      idx_ref[pl.ds(g * _L, _L)] = p

  def base_of(slot):
    return (wid + slot * _NW) * _CHUNK

  for b in range(3):
    compute_idx(base_of(b), idxs[b])
    pltpu.make_async_copy(
        emb_hbm.at[pl.ds(base_of(b), _CHUNK)], embs[b], ses[b]).start()
  pltpu.make_async_copy(pos_hbm.at[idxs[0]], poss[0], sps[0]).start()

  def tri_body(i, carry):
    for b in range(3):
      slot = 3 * i + b
      bn = (b + 1) % 3

      @pl.when(slot < nloc)
      def _do(slot=slot, b=b, bn=bn):
        base = base_of(slot)
        pltpu.make_async_copy(
            emb_hbm.at[pl.ds(base, _CHUNK)], embs[b], ses[b]).wait()
        pltpu.make_async_copy(pos_hbm.at[idxs[b]], poss[b], sps[b]).wait()

        def row_body(r, c2):
          for k in range(_D // _L):
            sl = pl.ds(k * _L, _L)
            plsc.addupdate(poss[b].at[r, sl], embs[b][r, sl] * _ALPHA)
          return c2
        lax.fori_loop(0, _CHUNK, row_body, 0)

        pltpu.make_async_copy(
            poss[b], out_hbm.at[pl.ds(base, _CHUNK)], sos[b]).start()

        @pl.when(slot + 3 < nloc)
        def _prefetch():
          base3 = base + 3 * _NW * _CHUNK
          compute_idx(base3, idxs[b])
          pltpu.make_async_copy(
              emb_hbm.at[pl.ds(base3, _CHUNK)], embs[b], ses[b]).start()

        @pl.when(slot + 1 < nloc)
        def _gather_next():
          @pl.when(slot >= 2)
          def _wait_prev_store():
            prev = base_of(slot - 2)
            pltpu.make_async_copy(
                poss[bn], out_hbm.at[pl.ds(prev, _CHUNK)], sos[bn]).wait()
          pltpu.make_async_copy(pos_hbm.at[idxs[bn]], poss[bn], sps[bn]).start()
    return carry

  lax.fori_loop(0, (_NMAX + 2) // 3, tri_body, 0)

  for b in range(3):
    pltpu.make_async_copy(poss[b], out_hbm.at[pl.ds(0, _CHUNK)], sos[b]).wait()


def kernel(max_seq_len, seq_lengths, seq_offsets, seq_embeddings,
           num_targets, pos_weight):
  high = jnp.minimum(seq_lengths - num_targets, _NPOS - 1).astype(jnp.int32)
  meta = jnp.concatenate([
      jnp.broadcast_to(seq_offsets[:_B, None].astype(jnp.int32), (_B, _L)),
      jnp.broadcast_to(high[:, None], (_B, _L)),
  ], axis=0)

  f = pl.kernel(
      _body,
      out_type=jax.ShapeDtypeStruct((_TOTAL, _D), jnp.float32),
      mesh=plsc.VectorSubcoreMesh(core_axis_name="c", subcore_axis_name="s"),
      scratch_types=[
          pltpu.VMEM((2 * _B, _L), jnp.int32),
          pltpu.VMEM((_CHUNK,), jnp.int32),
          pltpu.VMEM((_CHUNK,), jnp.int32),
          pltpu.VMEM((_CHUNK,), jnp.int32),
          pltpu.VMEM((_CHUNK, _D), jnp.float32),
          pltpu.VMEM((_CHUNK, _D), jnp.float32),
          pltpu.VMEM((_CHUNK, _D), jnp.float32),
          pltpu.VMEM((_CHUNK, _D), jnp.float32),
          pltpu.VMEM((_CHUNK, _D), jnp.float32),
          pltpu.VMEM((_CHUNK, _D), jnp.float32),
          pltpu.SemaphoreType.DMA,
          pltpu.SemaphoreType.DMA,
          pltpu.SemaphoreType.DMA,
          pltpu.SemaphoreType.DMA,
          pltpu.SemaphoreType.DMA,
          pltpu.SemaphoreType.DMA,
          pltpu.SemaphoreType.DMA,
          pltpu.SemaphoreType.DMA,
          pltpu.SemaphoreType.DMA,
      ],
  )
  return f(meta, seq_embeddings, pos_weight)


# speedup vs baseline: 1.2372x; 1.2372x over previous
import jax
import jax.numpy as jnp
from jax import lax
from jax.experimental import pallas as pl
from jax.experimental.pallas import tpu as pltpu
from jax.experimental.pallas import tpu_sc as plsc

_B = 8
_D = 512
_TOTAL = 15488
_NPOS = 8192
_ALPHA = float(_D) ** 0.5
_L = 16
_CHUNK = 32
_NCHUNKS = _TOTAL // _CHUNK
_NW = 32
_NMAX = -(-_NCHUNKS // _NW)


def _body(meta_hbm, emb_hbm, pos_hbm, out_hbm,
          meta_v, idx0, idx1, idx2, idx3, emb0, emb1,
          acc0, acc1, acc2, acc3,
          se0, se1, sp0, sp1, sp2, sp3, so0, so1, so2, so3):
  cid = lax.axis_index("c")
  sid = lax.axis_index("s")
  wid = sid * 2 + cid

  pltpu.sync_copy(meta_hbm, meta_v)
  off = [meta_v[b, :] for b in range(_B)]
  high = [meta_v[_B + b, :] for b in range(_B)]
  lanes = lax.iota(jnp.int32, _L)

  nloc = (_NCHUNKS - wid + _NW - 1) // _NW

  idxs = [idx0, idx1, idx2, idx3]
  embs = [emb0, emb1]
  accs = [acc0, acc1, acc2, acc3]
  ses = [se0, se1]
  sps = [sp0, sp1, sp2, sp3]
  sos = [so0, so1, so2, so3]

  def compute_idx(base, idx_ref):
    for g in range(_CHUNK // _L):
      t = base + g * _L + lanes
      off_s = off[0]
      high_s = high[0]
      for s in range(1, _B):
        m = t >= off[s]
        off_s = jnp.where(m, off[s], off_s)
        high_s = jnp.where(m, high[s], high_s)
      p = jnp.minimum(t - off_s, high_s)
      p = jnp.maximum(jnp.minimum(p, _NPOS - 1), 0)
      idx_ref[pl.ds(g * _L, _L)] = p

  def base_of(slot):
    return (wid + slot * _NW) * _CHUNK

  for s in range(2):
    compute_idx(base_of(s), idxs[s])
    pltpu.make_async_copy(
        emb_hbm.at[pl.ds(base_of(s), _CHUNK)], embs[s], ses[s]).start()
    pltpu.make_async_copy(pos_hbm.at[idxs[s]], accs[s], sps[s]).start()

  def quad_body(i, carry):
    for q in range(4):
      slot = 4 * i + q
      b2 = q % 2
      b4 = q
      b4n = (q + 2) % 4

      @pl.when(slot < nloc)
      def _do(slot=slot, b2=b2, b4=b4, b4n=b4n):
        base = base_of(slot)
        pltpu.make_async_copy(
            emb_hbm.at[pl.ds(base, _CHUNK)], embs[b2], ses[b2]).wait()
        pltpu.make_async_copy(pos_hbm.at[idxs[b4]], accs[b4], sps[b4]).wait()

        def row_body(r, c2):
          for k in range(_D // _L):
            sl = pl.ds(k * _L, _L)
            plsc.addupdate(accs[b4].at[r, sl], embs[b2][r, sl] * _ALPHA)
          return c2
        lax.fori_loop(0, _CHUNK, row_body, 0)

        pltpu.make_async_copy(
            accs[b4], out_hbm.at[pl.ds(base, _CHUNK)], sos[b4]).start()

        @pl.when(slot + 2 < nloc)
        def _prefetch():
          base2 = base + 2 * _NW * _CHUNK
          compute_idx(base2, idxs[b4n])
          pltpu.make_async_copy(
              emb_hbm.at[pl.ds(base2, _CHUNK)], embs[b2], ses[b2]).start()

          @pl.when(slot >= 2)
          def _wait_prev_store():
            prev = base_of(slot - 2)
            pltpu.make_async_copy(
                accs[b4n], out_hbm.at[pl.ds(prev, _CHUNK)], sos[b4n]).wait()
          pltpu.make_async_copy(
              pos_hbm.at[idxs[b4n]], accs[b4n], sps[b4n]).start()
    return carry

  lax.fori_loop(0, (_NMAX + 3) // 4, quad_body, 0)

  for b in range(4):
    pltpu.make_async_copy(accs[b], out_hbm.at[pl.ds(0, _CHUNK)], sos[b]).wait()


def kernel(max_seq_len, seq_lengths, seq_offsets, seq_embeddings,
           num_targets, pos_weight):
  high = jnp.minimum(seq_lengths - num_targets, _NPOS - 1).astype(jnp.int32)
  meta = jnp.concatenate([
      jnp.broadcast_to(seq_offsets[:_B, None].astype(jnp.int32), (_B, _L)),
      jnp.broadcast_to(high[:, None], (_B, _L)),
  ], axis=0)

  f = pl.kernel(
      _body,
      out_type=jax.ShapeDtypeStruct((_TOTAL, _D), jnp.float32),
      mesh=plsc.VectorSubcoreMesh(core_axis_name="c", subcore_axis_name="s"),
      scratch_types=[
          pltpu.VMEM((2 * _B, _L), jnp.int32),
          pltpu.VMEM((_CHUNK,), jnp.int32),
          pltpu.VMEM((_CHUNK,), jnp.int32),
          pltpu.VMEM((_CHUNK,), jnp.int32),
          pltpu.VMEM((_CHUNK,), jnp.int32),
          pltpu.VMEM((_CHUNK, _D), jnp.float32),
          pltpu.VMEM((_CHUNK, _D), jnp.float32),
          pltpu.VMEM((_CHUNK, _D), jnp.float32),
          pltpu.VMEM((_CHUNK, _D), jnp.float32),
          pltpu.VMEM((_CHUNK, _D), jnp.float32),
          pltpu.VMEM((_CHUNK, _D), jnp.float32),
          pltpu.SemaphoreType.DMA,
          pltpu.SemaphoreType.DMA,
          pltpu.SemaphoreType.DMA,
          pltpu.SemaphoreType.DMA,
          pltpu.SemaphoreType.DMA,
          pltpu.SemaphoreType.DMA,
          pltpu.SemaphoreType.DMA,
          pltpu.SemaphoreType.DMA,
          pltpu.SemaphoreType.DMA,
          pltpu.SemaphoreType.DMA,
      ],
  )
  return f(meta, seq_embeddings, pos_weight)
